# Initial kernel scaffold; baseline (speedup 1.0000x reference)
import jax
import jax.numpy as jnp
from jax.experimental import pallas as pl


def kernel(x, edge_index, batch, graph_features, W1, a1_src, a1_dst, b1, W2, a2_src, a2_dst, b2, Wl, bl):
    raise NotImplementedError("write your pallas kernel here")


# speedup vs baseline: 16.2682x; 16.2682x over previous
import functools

import jax
import jax.numpy as jnp
from jax import lax
from jax.experimental import pallas as pl
from jax.experimental.pallas import tpu as pltpu
from jax.experimental.pallas import tpu_sc as plsc

N = 10000
D = 128
NB = 64
NSEG = 10112
EE = 330000
NW = 32
T = 10336
EEP = T * NW
NCH = T // 16
STRIPE = NSEG // 16

_mesh = plsc.VectorSubcoreMesh(core_axis_name="c", subcore_axis_name="s")


@functools.partial(
    pl.kernel,
    out_type=(
        jax.ShapeDtypeStruct((EEP,), jnp.float32),
        jax.ShapeDtypeStruct((NW, NSEG), jnp.float32),
    ),
    mesh=_mesh,
    compiler_params=pltpu.CompilerParams(needs_layout_passes=False),
    scratch_types=[
        pltpu.VMEM((T,), jnp.int32),
        pltpu.VMEM((T,), jnp.int32),
        pltpu.VMEM((NSEG,), jnp.float32),
        pltpu.VMEM((NSEG,), jnp.float32),
        pltpu.VMEM((T,), jnp.float32),
        pltpu.VMEM((NSEG,), jnp.float32),
    ],
)
def _sc_edge_logits(src_hbm, dst_hbm, asad_hbm, ee_hbm, sparts_hbm,
                    src_v, dst_v, as_v, ad_v, ee_v, spart_v):
    c = lax.axis_index("c")
    s = lax.axis_index("s")
    wid = c * 16 + s
    base = wid * T
    pltpu.sync_copy(src_hbm.at[pl.ds(base, T)], src_v)
    pltpu.sync_copy(dst_hbm.at[pl.ds(base, T)], dst_v)
    pltpu.sync_copy(asad_hbm.at[0], as_v)
    pltpu.sync_copy(asad_hbm.at[1], ad_v)

    zero16 = jnp.zeros((16,), jnp.float32)

    def zero_body(i, carry):
        spart_v[pl.ds(i * 16, 16)] = zero16
        return carry

    lax.fori_loop(0, NSEG // 16, zero_body, 0)

    def edge_body(i, carry):
        off = i * 16
        sv = src_v[pl.ds(off, 16)]
        dv = dst_v[pl.ds(off, 16)]
        z = plsc.load_gather(as_v, [sv]) + plsc.load_gather(ad_v, [dv])
        e = jnp.where(z >= 0.0, z, z * jnp.float32(0.2))
        ee = jnp.exp(e)
        ee_v[pl.ds(off, 16)] = ee
        plsc.addupdate_scatter(spart_v, [dv], ee)
        return carry

    lax.fori_loop(0, NCH, edge_body, 0)
    pltpu.sync_copy(ee_v, ee_hbm.at[pl.ds(base, T)])
    pltpu.sync_copy(spart_v, sparts_hbm.at[wid])


@functools.partial(
    pl.kernel,
    out_type=jax.ShapeDtypeStruct((2, NSEG, D), jnp.float32),
    mesh=_mesh,
    compiler_params=pltpu.CompilerParams(needs_layout_passes=False),
    scratch_types=[
        pltpu.VMEM((T,), jnp.int32),
        pltpu.VMEM((T,), jnp.int32),
        pltpu.VMEM((T,), jnp.float32),
        pltpu.VMEM((NSEG,), jnp.float32),
        pltpu.VMEM((16, D), jnp.float32),
        pltpu.VMEM((16, D), jnp.float32),
        pltpu.VMEM((16,), jnp.int32),
        pltpu.VMEM((16,), jnp.int32),
        pltpu.VMEM_SHARED((NSEG, D), jnp.float32),
        pltpu.SemaphoreType.DMA,
    ],
)
def _sc_scatter_rows(src_hbm, dst_hbm, ee_hbm, recip_hbm, h_hbm, parts_hbm,
                     src_v, dst_v, ee_v, recip_v, rows_v, scaled_v,
                     sidx_v, didx_v, acc_sp, sem):
    c = lax.axis_index("c")
    s = lax.axis_index("s")
    wid = c * 16 + s
    base = wid * T
    pltpu.sync_copy(src_hbm.at[pl.ds(base, T)], src_v)
    pltpu.sync_copy(dst_hbm.at[pl.ds(base, T)], dst_v)
    pltpu.sync_copy(ee_hbm.at[pl.ds(base, T)], ee_v)
    pltpu.sync_copy(recip_hbm.at[0], recip_v)

    zero16 = jnp.zeros((16,), jnp.float32)
    for r in range(16):
        for j in range(D // 16):
            scaled_v[r, pl.ds(j * 16, 16)] = zero16

    stripe_base = s * STRIPE

    def zero_body(i, carry):
        pltpu.sync_copy(scaled_v, acc_sp.at[pl.ds(stripe_base + i * 16, 16)])
        return carry

    lax.fori_loop(0, STRIPE // 16, zero_body, 0)
    pltpu.sync_copy(scaled_v.at[pl.ds(0, STRIPE % 16)],
                    acc_sp.at[pl.ds(stripe_base + (STRIPE // 16) * 16,
                                    STRIPE % 16)])
    plsc.subcore_barrier()

    def edge_body(i, carry):
        off = i * 16
        sidx_v[...] = src_v[pl.ds(off, 16)]
        dv = dst_v[pl.ds(off, 16)]
        didx_v[...] = dv
        pltpu.async_copy(h_hbm.at[sidx_v], rows_v, sem).wait()
        alpha = ee_v[pl.ds(off, 16)] * plsc.load_gather(recip_v, [dv])
        for r in range(16):
            w = alpha[r]
            for j in range(D // 16):
                scaled_v[r, pl.ds(j * 16, 16)] = (
                    rows_v[r, pl.ds(j * 16, 16)] * w)
        pltpu.sync_copy(scaled_v, acc_sp.at[didx_v], add=True)
        return carry

    lax.fori_loop(0, NCH, edge_body, 0)
    plsc.subcore_barrier()
    pltpu.sync_copy(acc_sp.at[pl.ds(stripe_base, STRIPE)],
                    parts_hbm.at[c, pl.ds(stripe_base, STRIPE)])


def _mm_body(x_ref, w_ref, a_ref, h_ref, asad_ref):
    h = lax.dot_general(x_ref[...], w_ref[...], (((1,), (0,)), ((), ())),
                        preferred_element_type=jnp.float32)
    h_ref[...] = h
    asad_ref[...] = lax.dot_general(a_ref[...], h, (((1,), (1,)), ((), ())),
                                    preferred_element_type=jnp.float32)[None]


def _mm_call(x_p, W, A):
    return pl.pallas_call(
        _mm_body,
        grid=(8,),
        in_specs=[
            pl.BlockSpec((NSEG // 8, D), lambda i: (i, 0)),
            pl.BlockSpec((D, D), lambda i: (0, 0)),
            pl.BlockSpec((2, D), lambda i: (0, 0)),
        ],
        out_specs=[
            pl.BlockSpec((NSEG // 8, D), lambda i: (i, 0)),
            pl.BlockSpec((1, 2, NSEG // 8), lambda i: (i, 0, 0)),
        ],
        out_shape=[
            jax.ShapeDtypeStruct((NSEG, D), jnp.float32),
            jax.ShapeDtypeStruct((8, 2, NSEG // 8), jnp.float32),
        ],
    )(x_p, W, A)


def _recip_body(s_ref, r_ref):
    r_ref[...] = 1.0 / jnp.sum(s_ref[...], axis=0, keepdims=True)


def _recip_call(sparts):
    return pl.pallas_call(
        _recip_body,
        out_shape=jax.ShapeDtypeStruct((1, NSEG), jnp.float32),
    )(sparts)


def _ep_body(p_ref, b_ref, o_ref):
    o_ref[...] = jnp.maximum(p_ref[0] + p_ref[1] + b_ref[...], 0.0)


def _ep_call(parts, brow):
    return pl.pallas_call(
        _ep_body,
        grid=(8,),
        in_specs=[
            pl.BlockSpec((2, NSEG // 8, D), lambda i: (0, i, 0)),
            pl.BlockSpec((1, D), lambda i: (0, 0)),
        ],
        out_specs=pl.BlockSpec((NSEG // 8, D), lambda i: (i, 0)),
        out_shape=jax.ShapeDtypeStruct((NSEG, D), jnp.float32),
    )(parts, brow)


def _pool_body(g_ref, batch_ref, gf_ref, wl_ref, bl_ref, o_ref):
    oh = (batch_ref[...] ==
          lax.broadcasted_iota(jnp.int32, (NB, NSEG), 0)).astype(jnp.float32)
    sums = lax.dot_general(oh, g_ref[...], (((1,), (0,)), ((), ())),
                           preferred_element_type=jnp.float32)
    counts = jnp.sum(oh, axis=1, keepdims=True)
    pooled = sums / jnp.maximum(counts, 1.0)
    wl = wl_ref[...]
    o_ref[...] = (
        lax.dot_general(pooled, wl[0:D], (((1,), (0,)), ((), ())),
                        preferred_element_type=jnp.float32)
        + lax.dot_general(gf_ref[...], wl[D:], (((1,), (0,)), ((), ())),
                          preferred_element_type=jnp.float32)
        + bl_ref[...])


def _pool_call(g2, batch_p, gf, Wl, blrow):
    return pl.pallas_call(
        _pool_body,
        out_shape=jax.ShapeDtypeStruct((NB, 10), jnp.float32),
    )(g2, batch_p, gf, Wl, blrow)


def _gat_layer(src, dst, x_p, W, A, brow):
    h, asad3 = _mm_call(x_p, W, A)
    asad = jnp.transpose(asad3, (1, 0, 2)).reshape(2, NSEG)
    ee, sparts = _sc_edge_logits(src, dst, asad)
    recip = _recip_call(sparts)
    parts = _sc_scatter_rows(src, dst, ee, recip, h)
    return _ep_call(parts, brow)


def kernel(x, edge_index, batch, graph_features,
           W1, a1_src, a1_dst, b1, W2, a2_src, a2_dst, b2, Wl, bl):
    loop = jnp.arange(N, dtype=jnp.int32)
    pad = EEP - EE
    src = jnp.concatenate([edge_index[0], loop,
                           jnp.zeros((pad,), jnp.int32)])
    dst = jnp.concatenate([edge_index[1], loop,
                           jnp.full((pad,), N, jnp.int32)])
    x_p = jnp.pad(x, ((0, NSEG - N), (0, 0)))
    batch_p = jnp.pad(batch, (0, NSEG - N),
                      constant_values=NB).reshape(1, NSEG)
    A1 = jnp.stack([a1_src, a1_dst])
    A2 = jnp.stack([a2_src, a2_dst])

    g1 = _gat_layer(src, dst, x_p, W1, A1, b1.reshape(1, D))
    g2 = _gat_layer(src, dst, g1, W2, A2, b2.reshape(1, D))
    return _pool_call(g2, batch_p, graph_features, Wl, bl.reshape(1, 10))
